# baseline (speedup 1.0000x reference)
import functools

import jax
import jax.numpy as jnp
from jax import lax
from jax.experimental import pallas as pl
from jax.experimental.pallas import tpu as pltpu
from jax.experimental.pallas import tpu_sc as plsc

L = 16
NC = 2
NS = 16
NW = NC * NS


def _build_sc_kernel(B, U, F, NI, FPP, D):
    PB = B // NW
    NG = PB // L
    KROWS = (U * F) // L

    mesh = plsc.VectorSubcoreMesh(core_axis_name="c", subcore_axis_name="s")

    @functools.partial(
        pl.kernel,
        out_type=jax.ShapeDtypeStruct((B,), jnp.float32),
        mesh=mesh,
        compiler_params=pltpu.CompilerParams(
            needs_layout_passes=False, use_tc_tiling_on_sc=False),
        scratch_types=[
            pltpu.VMEM((PB,), jnp.int32),
            pltpu.VMEM((PB,), jnp.int32),
            pltpu.VMEM((FPP, PB), jnp.int32),
            pltpu.VMEM((FPP, PB), jnp.int32),
            pltpu.VMEM((FPP, PB), jnp.int32),
            pltpu.VMEM((FPP, PB), jnp.int32),
            pltpu.VMEM((PB, L), jnp.float32),
            pltpu.VMEM((FPP, PB, L), jnp.float32),
            pltpu.VMEM((FPP, PB, L), jnp.float32),
            pltpu.VMEM((NI,), jnp.float32),
            pltpu.VMEM((PB,), jnp.float32),
            pltpu.SemaphoreType.DMA,
        ],
    )
    def sck(user_h, item_h, hp_h, gp_h, kf_h, ib_h, cf_h, out_h,
            user_v, item_v, cidx_v, gidx_v, krow_v, koff_v,
            hrows, grows, krows, ib_v, out_v, sem):
        wid = lax.axis_index("s") * NC + lax.axis_index("c")
        base = wid * PB
        pltpu.sync_copy(user_h.at[pl.ds(base, PB)], user_v)
        pltpu.sync_copy(item_h.at[pl.ds(base, PB)], item_v)
        ib_desc = pltpu.async_copy(ib_h, ib_v, sem)

        iota = lax.iota(jnp.int32, L)
        for g in range(NG):
            sl = pl.ds(g * L, L)
            cbase = (user_v[sl] * NI + item_v[sl]) * FPP
            for j in range(FPP):
                cidx_v[j, sl] = cbase + j
        cdescs = [pltpu.async_copy(cf_h.at[cidx_v.at[j]], gidx_v.at[j], sem)
                  for j in range(FPP)]
        for dsc in cdescs:
            dsc.wait()

        for g in range(NG):
            sl = pl.ds(g * L, L)
            u = user_v[sl]
            for j in range(FPP):
                f = gidx_v[j, sl]
                kidx = u * F + f
                krow_v[j, sl] = lax.shift_right_logical(kidx, 4)
                koff_v[j, sl] = lax.bitwise_and(kidx, L - 1)

        descs = [pltpu.async_copy(hp_h.at[user_v], hrows, sem)]
        for j in range(FPP):
            descs.append(pltpu.async_copy(gp_h.at[gidx_v.at[j]], grows.at[j], sem))
            descs.append(pltpu.async_copy(kf_h.at[krow_v.at[j]], krows.at[j], sem))
        for dsc in descs:
            dsc.wait()
        ib_desc.wait()

        for g in range(NG):
            sl = pl.ds(g * L, L)
            rows = iota + (g * L)
            hd = [plsc.load_gather(hrows, [rows, jnp.full((L,), d, jnp.int32)])
                  for d in range(D)]
            acc = plsc.load_gather(ib_v, [item_v[sl]])
            for j in range(FPP):
                jc = jnp.full((L,), j, jnp.int32)
                dot = jnp.zeros((L,), jnp.float32)
                for d in range(D):
                    gd = plsc.load_gather(
                        grows, [jc, rows, jnp.full((L,), d, jnp.int32)])
                    dot = dot + hd[d] * gd
                fb = plsc.load_gather(
                    grows, [jc, rows, jnp.full((L,), D, jnp.int32)])
                kv = plsc.load_gather(krows, [jc, rows, koff_v[j, sl]])
                acc = acc + kv * (dot + fb)
            out_v[sl] = acc
        pltpu.sync_copy(out_v, out_h.at[pl.ds(base, PB)])

    return sck


def kernel(user, item, H, G, K, F_B, I_B, C):
    B = user.shape[0]
    U, F = K.shape
    NI = I_B.shape[0]
    FPP = C.shape[2]
    D = H.shape[1]
    Hp = jnp.pad(H, ((0, 0), (0, L - D)))
    Gp = jnp.concatenate(
        [G, F_B[:, None], jnp.zeros((F, L - D - 1), jnp.float32)], axis=1)
    Kf = K.reshape((U * F) // L, L)
    Cf = C.reshape(U * NI * FPP)
    sck = _build_sc_kernel(B, U, F, NI, FPP, D)
    return sck(user, item, Hp, Gp, Kf, I_B, Cf)


# speedup vs baseline: 24.4609x; 24.4609x over previous
import functools

import jax
import jax.numpy as jnp
from jax import lax
from jax.experimental import pallas as pl
from jax.experimental.pallas import tpu as pltpu
from jax.experimental.pallas import tpu_sc as plsc

L = 16
NC = 2
NS = 16
NW = NC * NS
GW = 128


def _build_sc_kernel(B, U, F, NI, FPP, D):
    PB = B // NW
    NG = PB // L

    mesh = plsc.VectorSubcoreMesh(core_axis_name="c", subcore_axis_name="s")

    @functools.partial(
        pl.kernel,
        out_type=jax.ShapeDtypeStruct((B,), jnp.float32),
        mesh=mesh,
        compiler_params=pltpu.CompilerParams(
            needs_layout_passes=False, use_tc_tiling_on_sc=False),
        scratch_types=[
            pltpu.VMEM((PB,), jnp.int32),
            pltpu.VMEM((PB,), jnp.int32),
            pltpu.VMEM((FPP, PB), jnp.int32),
            pltpu.VMEM((FPP, PB), jnp.int32),
            pltpu.VMEM((FPP, PB), jnp.int32),
            pltpu.VMEM((FPP, PB), jnp.float32),
            pltpu.VMEM((PB, GW), jnp.float32),
            pltpu.VMEM((FPP, PB, GW), jnp.float32),
            pltpu.VMEM((NI,), jnp.float32),
            pltpu.VMEM((PB,), jnp.float32),
            pltpu.SemaphoreType.DMA,
        ],
    )
    def sck(user_h, item_h, hp_h, gp_h, k1_h, ib_h, cf_h, out_h,
            user_v, item_v, cidx_v, gidx_v, kidx_v, kval_v,
            hrows, grows, ib_v, out_v, sem):
        wid = lax.axis_index("s") * NC + lax.axis_index("c")
        base = wid * PB
        pltpu.sync_copy(user_h.at[pl.ds(base, PB)], user_v)
        pltpu.sync_copy(item_h.at[pl.ds(base, PB)], item_v)
        ib_desc = pltpu.async_copy(ib_h, ib_v, sem)

        iota = lax.iota(jnp.int32, L)
        for g in range(NG):
            sl = pl.ds(g * L, L)
            u = user_v[sl]
            cbase = item_v[sl] * (FPP * U) + u
            for j in range(FPP):
                cidx_v[j, sl] = cbase + j * U
        cdescs = [pltpu.async_copy(cf_h.at[cidx_v.at[j]], gidx_v.at[j], sem)
                  for j in range(FPP)]
        for dsc in cdescs:
            dsc.wait()

        for g in range(NG):
            sl = pl.ds(g * L, L)
            u = user_v[sl]
            for j in range(FPP):
                kidx_v[j, sl] = gidx_v[j, sl] * U + u

        descs = [pltpu.async_copy(hp_h.at[user_v], hrows, sem)]
        for j in range(FPP):
            descs.append(pltpu.async_copy(gp_h.at[gidx_v.at[j]], grows.at[j], sem))
            descs.append(pltpu.async_copy(k1_h.at[kidx_v.at[j]], kval_v.at[j], sem))
        for dsc in descs:
            dsc.wait()
        ib_desc.wait()

        for g in range(NG):
            sl = pl.ds(g * L, L)
            rows = iota + (g * L)
            hd = [plsc.load_gather(hrows, [rows, jnp.full((L,), d, jnp.int32)])
                  for d in range(D)]
            acc = plsc.load_gather(ib_v, [item_v[sl]])
            for j in range(FPP):
                jc = jnp.full((L,), j, jnp.int32)
                dot = jnp.zeros((L,), jnp.float32)
                for d in range(D):
                    gd = plsc.load_gather(
                        grows, [jc, rows, jnp.full((L,), d, jnp.int32)])
                    dot = dot + hd[d] * gd
                fb = plsc.load_gather(
                    grows, [jc, rows, jnp.full((L,), D, jnp.int32)])
                acc = acc + kval_v[j, sl] * (dot + fb)
            out_v[sl] = acc
        pltpu.sync_copy(out_v, out_h.at[pl.ds(base, PB)])

    return sck


def kernel(user, item, H, G, K, F_B, I_B, C):
    B = user.shape[0]
    U, F = K.shape
    NI = I_B.shape[0]
    FPP = C.shape[2]
    D = H.shape[1]
    Hp = jnp.pad(H, ((0, 0), (0, GW - D)))
    Gp = jnp.concatenate(
        [G, F_B[:, None], jnp.zeros((F, GW - D - 1), jnp.float32)], axis=1)
    K1 = K.T.reshape(U * F)
    Cf = C.transpose(1, 2, 0).reshape(NI * FPP * U)
    K1, Cf = lax.optimization_barrier((K1, Cf))
    sck = _build_sc_kernel(B, U, F, NI, FPP, D)
    return sck(user, item, Hp, Gp, K1, I_B, Cf)
